# Initial kernel scaffold; baseline (speedup 1.0000x reference)
import jax
import jax.numpy as jnp
from jax.experimental import pallas as pl


def kernel(x, token_table, pos_table):
    raise NotImplementedError("write your pallas kernel here")


# speedup vs baseline: 1.4273x; 1.4273x over previous
import functools

import jax
import jax.numpy as jnp
from jax import lax
from jax.experimental import pallas as pl
from jax.experimental.pallas import tpu as pltpu
from jax.experimental.pallas import tpu_sc as plsc


def _emb_kernel(N, E, L, NC, NS, CH):
    NW = NC * NS
    rows_per_w = N // NW
    n_ch = rows_per_w // CH
    reps = CH // L
    mesh = plsc.VectorSubcoreMesh(core_axis_name="c", subcore_axis_name="s")

    @functools.partial(
        pl.kernel,
        mesh=mesh,
        compiler_params=pltpu.CompilerParams(use_tc_tiling_on_sc=False),
        out_type=jax.ShapeDtypeStruct((N, E), jnp.float32),
        scratch_types=[
            pltpu.VMEM((CH,), jnp.int32),
            pltpu.VMEM((CH, E), jnp.float32),
            pltpu.VMEM((L, E), jnp.float32),
            pltpu.SemaphoreType.DMA,
        ],
    )
    def k(x_hbm, tok_hbm, pos_hbm, out_hbm, idx_v, rows_v, pos_v, sem):
        wid = lax.axis_index("s") * NC + lax.axis_index("c")
        base = wid * rows_per_w
        pltpu.sync_copy(pos_hbm, pos_v)

        def chunk_body(c, carry):
            cb = base + c * CH
            pltpu.sync_copy(x_hbm.at[pl.ds(cb, CH)], idx_v)
            pltpu.async_copy(tok_hbm.at[idx_v], rows_v, sem).wait()

            def pos_body(p, carry2):
                for h in range(E // 16):
                    cs = pl.ds(h * 16, 16)
                    pv = pos_v[p, cs]
                    for j in range(reps):
                        r = j * L + p
                        rows_v[r, cs] = rows_v[r, cs] + pv
                return carry2

            lax.fori_loop(0, L, pos_body, 0, unroll=2)
            pltpu.sync_copy(rows_v, out_hbm.at[pl.ds(cb, CH)])
            return carry

        lax.fori_loop(0, n_ch, chunk_body, 0)

    return k


def kernel(x, token_table, pos_table):
    B, L = x.shape
    V, E = token_table.shape
    N = B * L
    x_flat = x.reshape(N).astype(jnp.int32)
    k = _emb_kernel(N, E, L, NC=2, NS=16, CH=8 * L)
    out = k(x_flat, token_table, pos_table)
    return out.reshape(B, L, E)


# speedup vs baseline: 1.9161x; 1.3425x over previous
import functools

import jax
import jax.numpy as jnp
from jax import lax
from jax.experimental import pallas as pl
from jax.experimental.pallas import tpu as pltpu
from jax.experimental.pallas import tpu_sc as plsc


def _emb_kernel(N, E, L, NC, NS, CH):
    NW = NC * NS
    rows_per_w = N // NW
    n_ch = rows_per_w // CH
    reps = CH // L
    mesh = plsc.VectorSubcoreMesh(core_axis_name="c", subcore_axis_name="s")

    @functools.partial(
        pl.kernel,
        mesh=mesh,
        compiler_params=pltpu.CompilerParams(use_tc_tiling_on_sc=False),
        out_type=jax.ShapeDtypeStruct((N, 128), jnp.float32),
        scratch_types=[
            pltpu.VMEM((CH,), jnp.int32),
            pltpu.VMEM((CH, E), jnp.float32),
            pltpu.VMEM((L, E), jnp.float32),
            pltpu.SemaphoreType.DMA,
        ],
    )
    def k(x_hbm, tok_hbm, pos_hbm, out_hbm, idx_v, rows_v, pos_v, sem):
        wid = lax.axis_index("s") * NC + lax.axis_index("c")
        base = wid * rows_per_w
        pltpu.sync_copy(pos_hbm, pos_v)

        def chunk_body(c, carry):
            cb = base + c * CH
            pltpu.sync_copy(x_hbm.at[pl.ds(cb, CH)], idx_v)
            pltpu.async_copy(tok_hbm.at[idx_v], rows_v, sem).wait()

            def pos_body(p, carry2):
                for h in range(E // 16):
                    cs = pl.ds(h * 16, 16)
                    pv = pos_v[p, cs]
                    for j in range(reps):
                        r = j * L + p
                        rows_v[r, cs] = rows_v[r, cs] + pv
                return carry2

            lax.fori_loop(0, L, pos_body, 0, unroll=2)
            pltpu.sync_copy(rows_v, out_hbm.at[pl.ds(cb, CH), pl.ds(0, E)])
            return carry

        lax.fori_loop(0, n_ch, chunk_body, 0)

    return k


def kernel(x, token_table, pos_table):
    B, L = x.shape
    V, E = token_table.shape
    N = B * L
    x_flat = x.reshape(N).astype(jnp.int32)
    k = _emb_kernel(N, E, L, NC=2, NS=16, CH=8 * L)
    out = k(x_flat, token_table, pos_table)
    return out[:, :E].reshape(B, L, E)
